# Initial kernel scaffold; baseline (speedup 1.0000x reference)
import jax
import jax.numpy as jnp
from jax.experimental import pallas as pl


def kernel(log_h, y):
    raise NotImplementedError("write your pallas kernel here")


# speedup vs baseline: 5.4510x; 5.4510x over previous
import functools

import jax
import jax.numpy as jnp
from jax import lax
from jax.experimental import pallas as pl
from jax.experimental.pallas import tpu as pltpu
from jax.experimental.pallas import tpu_sc as plsc

N = 65536
NB = 1024
L = 16
NC, NS = 2, 16
NW = NC * NS
K = N // NW
NV = K // L

_mesh = plsc.VectorSubcoreMesh(core_axis_name="c", subcore_axis_name="s")


@functools.partial(
    pl.kernel,
    mesh=_mesh,
    compiler_params=pltpu.CompilerParams(needs_layout_passes=False),
    out_type=[
        jax.ShapeDtypeStruct((NW, NB), jnp.float32),
        jax.ShapeDtypeStruct((NW, NB), jnp.float32),
        jax.ShapeDtypeStruct((NW, NB), jnp.float32),
        jax.ShapeDtypeStruct((NW, NB), jnp.int32),
        jax.ShapeDtypeStruct((NW, 2 * L), jnp.float32),
    ],
    scratch_types=[
        pltpu.VMEM((K,), jnp.float32),
        pltpu.VMEM((2 * K,), jnp.int32),
        pltpu.VMEM((K,), jnp.float32),
        pltpu.VMEM((NB,), jnp.float32),
        pltpu.VMEM((NB,), jnp.float32),
        pltpu.VMEM((NB,), jnp.float32),
        pltpu.VMEM((NB,), jnp.int32),
        pltpu.VMEM((2 * L,), jnp.float32),
    ],
)
def _sc_hist(lh_hbm, y_hbm, E_out, c_out, P_out, jx_out, nums_out,
             lh_v, y_v, e_v, E_v, c_v, P_v, jx_v, nums_v):
    wid = lax.axis_index("s") * NC + lax.axis_index("c")
    base = wid * K
    pltpu.sync_copy(lh_hbm.at[pl.ds(base, K)], lh_v)
    pltpu.sync_copy(y_hbm.at[pl.ds(2 * base, 2 * K)], y_v)

    zf = jnp.zeros((L,), jnp.float32)
    neg1 = jnp.full((L,), -1, jnp.int32)

    def zero_body(j, carry):
        sl = pl.ds(j * L, L)
        E_v[sl] = zf
        c_v[sl] = zf
        P_v[sl] = zf
        jx_v[sl] = neg1
        return carry

    lax.fori_loop(0, NB // L, zero_body, 0)
    nums_v[pl.ds(0, L)] = zf
    nums_v[pl.ds(L, L)] = zf

    lane = lax.iota(jnp.int32, L)
    onesf = jnp.ones((L,), jnp.float32)

    def pass1(j, carry):
        ns, nc = carry
        sl = pl.ds(j * L, L)
        iloc = j * L + lane
        lh = lh_v[sl]
        d = plsc.load_gather(y_v, [2 * iloc])
        ev = plsc.load_gather(y_v, [2 * iloc + 1])
        e = jnp.exp(lh)
        e_v[sl] = e
        evm = ev != 0
        plsc.addupdate_scatter(E_v, [d], e)
        plsc.addupdate_scatter(c_v, [d], onesf, mask=evm)
        plsc.store_scatter(jx_v, [d], iloc, mask=evm)
        ns = ns + jnp.where(evm, lh, 0.0)
        nc = nc + jnp.where(evm, 1.0, 0.0)
        return ns, nc

    ns, nc = lax.fori_loop(0, NV, pass1, (zf, zf))
    nums_v[pl.ds(0, L)] = ns
    nums_v[pl.ds(L, L)] = nc

    def pass2(j, carry):
        sl = pl.ds(j * L, L)
        iloc = j * L + lane
        d = plsc.load_gather(y_v, [2 * iloc])
        jm = plsc.load_gather(jx_v, [d])
        plsc.addupdate_scatter(P_v, [d], e_v[sl], mask=iloc <= jm)
        return carry

    lax.fori_loop(0, NV, pass2, 0)

    pltpu.sync_copy(E_v, E_out.at[wid])
    pltpu.sync_copy(c_v, c_out.at[wid])
    pltpu.sync_copy(P_v, P_out.at[wid])
    pltpu.sync_copy(jx_v, jx_out.at[wid])
    pltpu.sync_copy(nums_v, nums_out.at[wid])


def _tc_combine(E_ref, c_ref, P_ref, jx_ref, nums_ref, out_ref):
    E = E_ref[...]
    cmat = c_ref[...]
    Pm = P_ref[...]
    jx = jx_ref[...]
    hasb = jx >= 0
    has = hasb.astype(jnp.float32)
    ti = lax.broadcasted_iota(jnp.int32, (NW, NW), 0)
    tj = lax.broadcasted_iota(jnp.int32, (NW, NW), 1)
    strict_later = (tj > ti).astype(jnp.float32)
    suffix_cnt = jnp.dot(strict_later, has,
                         preferred_element_type=jnp.float32,
                         precision=lax.Precision.HIGHEST)
    any_later = suffix_cnt > 0.5
    Pd = jnp.sum(jnp.where(any_later, E, jnp.where(hasb, Pm, 0.0)),
                 axis=0, keepdims=True)
    Ed = jnp.sum(E, axis=0, keepdims=True)
    cd = jnp.sum(cmat, axis=0, keepdims=True)
    bi = lax.broadcasted_iota(jnp.int32, (NB, NB), 0)
    bj = lax.broadcasted_iota(jnp.int32, (NB, NB), 1)
    U = (bi > bj).astype(jnp.float32)
    Sgt = jnp.dot(Ed, U, preferred_element_type=jnp.float32,
                  precision=lax.Precision.HIGHEST)
    grp = cd > 0
    A = jnp.where(grp, Sgt + Pd, 1.0)
    den = (jnp.sum(jnp.where(grp, cd * jnp.log(A), 0.0))
           / jnp.sum(grp.astype(jnp.float32)))
    nums = nums_ref[...]
    num = jnp.sum(nums[:, 0:L]) / jnp.sum(nums[:, L:2 * L])
    out_ref[...] = jnp.reshape(den - num, (1, 1))


def kernel(log_h, y):
    log_h = log_h.reshape(-1)
    E, c, P, jx, nums = _sc_hist(log_h, y.reshape(-1))
    out = pl.pallas_call(
        _tc_combine,
        out_shape=jax.ShapeDtypeStruct((1, 1), jnp.float32),
    )(E, c, P, jx, nums)
    return out[0, 0]


# speedup vs baseline: 12.6072x; 2.3128x over previous
import functools

import jax
import jax.numpy as jnp
from jax import lax
from jax.experimental import pallas as pl
from jax.experimental.pallas import tpu as pltpu
from jax.experimental.pallas import tpu_sc as plsc

N = 65536
NB = 1024
L = 16
NC, NS = 2, 16
NW = NC * NS
K = N // NW
NV = K // L

_mesh = plsc.VectorSubcoreMesh(core_axis_name="c", subcore_axis_name="s")


@functools.partial(
    pl.kernel,
    mesh=_mesh,
    compiler_params=pltpu.CompilerParams(needs_layout_passes=False),
    out_type=[
        jax.ShapeDtypeStruct((NW, NB), jnp.float32),
        jax.ShapeDtypeStruct((NW, NB), jnp.float32),
        jax.ShapeDtypeStruct((NW, NB), jnp.float32),
        jax.ShapeDtypeStruct((NW, NB), jnp.int32),
        jax.ShapeDtypeStruct((NW, 2 * L), jnp.float32),
    ],
    scratch_types=[
        pltpu.VMEM((K,), jnp.float32),
        pltpu.VMEM((K,), jnp.int32),
        pltpu.VMEM((K,), jnp.int32),
        pltpu.VMEM((K,), jnp.float32),
        pltpu.VMEM((NB,), jnp.float32),
        pltpu.VMEM((NB,), jnp.float32),
        pltpu.VMEM((NB,), jnp.float32),
        pltpu.VMEM((NB,), jnp.int32),
        pltpu.VMEM((2 * L,), jnp.float32),
    ],
)
def _sc_hist(lh_hbm, dur_hbm, ev_hbm, E_out, c_out, P_out, jx_out, nums_out,
             lh_v, dur_v, ev_v, e_v, E_v, c_v, P_v, jx_v, nums_v):
    wid = lax.axis_index("s") * NC + lax.axis_index("c")
    base = wid * K
    pltpu.sync_copy(lh_hbm.at[pl.ds(base, K)], lh_v)
    pltpu.sync_copy(dur_hbm.at[pl.ds(base, K)], dur_v)
    pltpu.sync_copy(ev_hbm.at[pl.ds(base, K)], ev_v)

    zf = jnp.zeros((L,), jnp.float32)
    neg1 = jnp.full((L,), -1, jnp.int32)

    def zero_body(j, carry):
        sl = pl.ds(j * L, L)
        E_v[sl] = zf
        c_v[sl] = zf
        P_v[sl] = zf
        jx_v[sl] = neg1
        return carry

    lax.fori_loop(0, NB // L, zero_body, 0)
    nums_v[pl.ds(0, L)] = zf
    nums_v[pl.ds(L, L)] = zf

    lane = lax.iota(jnp.int32, L)
    onesf = jnp.ones((L,), jnp.float32)

    def pass1(j, carry):
        ns, nc = carry
        sl = pl.ds(j * L, L)
        iloc = j * L + lane
        lh = lh_v[sl]
        d = dur_v[sl]
        ev = ev_v[sl]
        e = jnp.exp(lh)
        e_v[sl] = e
        evm = ev != 0
        plsc.addupdate_scatter(E_v, [d], e)
        plsc.addupdate_scatter(c_v, [d], onesf, mask=evm)
        plsc.store_scatter(jx_v, [d], iloc, mask=evm)
        ns = ns + jnp.where(evm, lh, 0.0)
        nc = nc + jnp.where(evm, 1.0, 0.0)
        return ns, nc

    ns, nc = lax.fori_loop(0, NV, pass1, (zf, zf))
    nums_v[pl.ds(0, L)] = ns
    nums_v[pl.ds(L, L)] = nc

    def pass2(j, carry):
        sl = pl.ds(j * L, L)
        iloc = j * L + lane
        d = dur_v[sl]
        jm = plsc.load_gather(jx_v, [d])
        plsc.addupdate_scatter(P_v, [d], e_v[sl], mask=iloc <= jm)
        return carry

    lax.fori_loop(0, NV, pass2, 0)

    pltpu.sync_copy(E_v, E_out.at[wid])
    pltpu.sync_copy(c_v, c_out.at[wid])
    pltpu.sync_copy(P_v, P_out.at[wid])
    pltpu.sync_copy(jx_v, jx_out.at[wid])
    pltpu.sync_copy(nums_v, nums_out.at[wid])


def _tc_combine(E_ref, c_ref, P_ref, jx_ref, nums_ref, out_ref):
    E = E_ref[...]
    cmat = c_ref[...]
    Pm = P_ref[...]
    jx = jx_ref[...]
    hasb = jx >= 0
    has = hasb.astype(jnp.float32)
    ti = lax.broadcasted_iota(jnp.int32, (NW, NW), 0)
    tj = lax.broadcasted_iota(jnp.int32, (NW, NW), 1)
    strict_later = (tj > ti).astype(jnp.float32)
    suffix_cnt = jnp.dot(strict_later, has,
                         preferred_element_type=jnp.float32,
                         precision=lax.Precision.HIGHEST)
    any_later = suffix_cnt > 0.5
    Pd = jnp.sum(jnp.where(any_later, E, jnp.where(hasb, Pm, 0.0)),
                 axis=0, keepdims=True)
    Ed = jnp.sum(E, axis=0, keepdims=True)
    cd = jnp.sum(cmat, axis=0, keepdims=True)
    bi = lax.broadcasted_iota(jnp.int32, (NB, NB), 0)
    bj = lax.broadcasted_iota(jnp.int32, (NB, NB), 1)
    U = (bi > bj).astype(jnp.float32)
    Sgt = jnp.dot(Ed, U, preferred_element_type=jnp.float32,
                  precision=lax.Precision.HIGHEST)
    grp = cd > 0
    A = jnp.where(grp, Sgt + Pd, 1.0)
    den = (jnp.sum(jnp.where(grp, cd * jnp.log(A), 0.0))
           / jnp.sum(grp.astype(jnp.float32)))
    nums = nums_ref[...]
    num = jnp.sum(nums[:, 0:L]) / jnp.sum(nums[:, L:2 * L])
    out_ref[...] = jnp.reshape(den - num, (1, 1))


def kernel(log_h, y):
    log_h = log_h.reshape(-1)
    E, c, P, jx, nums = _sc_hist(log_h, y[:, 0], y[:, 1])
    out = pl.pallas_call(
        _tc_combine,
        out_shape=jax.ShapeDtypeStruct((1, 1), jnp.float32),
    )(E, c, P, jx, nums)
    return out[0, 0]


# speedup vs baseline: 13.7578x; 1.0913x over previous
import functools

import jax
import jax.numpy as jnp
from jax import lax
from jax.experimental import pallas as pl
from jax.experimental.pallas import tpu as pltpu
from jax.experimental.pallas import tpu_sc as plsc

N = 65536
NB = 1024
L = 16
NC, NS = 2, 16
NW = NC * NS
K = N // NW
NV = K // L

_mesh = plsc.VectorSubcoreMesh(core_axis_name="c", subcore_axis_name="s")


@functools.partial(
    pl.kernel,
    mesh=_mesh,
    compiler_params=pltpu.CompilerParams(needs_layout_passes=False),
    out_type=jax.ShapeDtypeStruct((NW, 4 * NB + 2 * L), jnp.float32),
    scratch_types=[
        pltpu.VMEM((K,), jnp.float32),
        pltpu.VMEM((K,), jnp.int32),
        pltpu.VMEM((K,), jnp.int32),
        pltpu.VMEM((NB,), jnp.float32),
        pltpu.VMEM((NB,), jnp.float32),
        pltpu.VMEM((NB,), jnp.float32),
        pltpu.VMEM((NB,), jnp.float32),
        pltpu.VMEM((2 * L,), jnp.float32),
        pltpu.SemaphoreType.DMA,
        pltpu.SemaphoreType.DMA,
    ],
)
def _sc_hist(lh_hbm, dur_hbm, ev_hbm, out_hbm,
             lh_v, dur_v, ev_v, E_v, c_v, P_v, jx_v, nums_v,
             sem_in, sem_out):
    wid = lax.axis_index("s") * NC + lax.axis_index("c")
    base = wid * K
    cp_lh = pltpu.async_copy(lh_hbm.at[pl.ds(base, K)], lh_v, sem_in)
    cp_d = pltpu.async_copy(dur_hbm.at[pl.ds(base, K)], dur_v, sem_in)
    cp_e = pltpu.async_copy(ev_hbm.at[pl.ds(base, K)], ev_v, sem_in)

    zf = jnp.zeros((L,), jnp.float32)
    neg1 = jnp.full((L,), -1.0, jnp.float32)

    def zero_body(j, carry):
        sl = pl.ds(j * L, L)
        E_v[sl] = zf
        c_v[sl] = zf
        P_v[sl] = zf
        jx_v[sl] = neg1
        return carry

    lax.fori_loop(0, NB // L, zero_body, 0)
    cp_lh.wait()
    cp_d.wait()
    cp_e.wait()

    lane = lax.iota(jnp.int32, L)
    lanef = lane.astype(jnp.float32)
    onesf = jnp.ones((L,), jnp.float32)
    U = 2

    def pass1(j, carry):
        ns, nc = carry
        for u in range(U):
            jj = U * j + u
            sl = pl.ds(jj * L, L)
            ilocf = jj * L + lanef
            lh = lh_v[sl]
            d = dur_v[sl]
            ev = ev_v[sl]
            e = jnp.exp(lh)
            evm = ev != 0
            plsc.addupdate_scatter(E_v, [d], e)
            plsc.addupdate_scatter(c_v, [d], onesf, mask=evm)
            plsc.store_scatter(jx_v, [d], ilocf, mask=evm)
            ns = ns + jnp.where(evm, lh, 0.0)
            nc = nc + jnp.where(evm, 1.0, 0.0)
        return ns, nc

    ns, nc = lax.fori_loop(0, NV // U, pass1, (zf, zf))
    nums_v[pl.ds(0, L)] = ns
    nums_v[pl.ds(L, L)] = nc

    def pass2(j, carry):
        for u in range(U):
            jj = U * j + u
            sl = pl.ds(jj * L, L)
            ilocf = jj * L + lanef
            d = dur_v[sl]
            jm = plsc.load_gather(jx_v, [d])
            plsc.addupdate_scatter(P_v, [d], jnp.exp(lh_v[sl]),
                                   mask=ilocf <= jm)
        return carry

    lax.fori_loop(0, NV // U, pass2, 0)

    row = out_hbm.at[wid]
    cps = [
        pltpu.async_copy(E_v, row.at[pl.ds(0, NB)], sem_out),
        pltpu.async_copy(c_v, row.at[pl.ds(NB, NB)], sem_out),
        pltpu.async_copy(P_v, row.at[pl.ds(2 * NB, NB)], sem_out),
        pltpu.async_copy(jx_v, row.at[pl.ds(3 * NB, NB)], sem_out),
        pltpu.async_copy(nums_v, row.at[pl.ds(4 * NB, 2 * L)], sem_out),
    ]
    for cp in cps:
        cp.wait()


def _tc_combine(packed_ref, out_ref):
    packed = packed_ref[...]
    E = packed[:, 0:NB]
    cmat = packed[:, NB:2 * NB]
    Pm = packed[:, 2 * NB:3 * NB]
    jx = packed[:, 3 * NB:4 * NB]
    hasb = jx >= 0
    has = hasb.astype(jnp.float32)
    ti = lax.broadcasted_iota(jnp.int32, (NW, NW), 0)
    tj = lax.broadcasted_iota(jnp.int32, (NW, NW), 1)
    strict_later = (tj > ti).astype(jnp.float32)
    suffix_cnt = jnp.dot(strict_later, has,
                         preferred_element_type=jnp.float32,
                         precision=lax.Precision.HIGHEST)
    any_later = suffix_cnt > 0.5
    Pd = jnp.sum(jnp.where(any_later, E, jnp.where(hasb, Pm, 0.0)),
                 axis=0).reshape(8, 128)
    Ed = jnp.sum(E, axis=0).reshape(8, 128)
    cd = jnp.sum(cmat, axis=0).reshape(8, 128)
    ci = lax.broadcasted_iota(jnp.int32, (128, 128), 0)
    cj = lax.broadcasted_iota(jnp.int32, (128, 128), 1)
    Uc = (ci > cj).astype(jnp.float32)
    within = jnp.dot(Ed, Uc, preferred_element_type=jnp.float32,
                     precision=lax.Precision.HIGHEST)
    ri = lax.broadcasted_iota(jnp.int32, (8, 8), 0)
    rj = lax.broadcasted_iota(jnp.int32, (8, 8), 1)
    Ur = (rj > ri).astype(jnp.float32)
    row_tot = jnp.sum(Ed, axis=1, keepdims=True)
    row_suf = jnp.dot(Ur, row_tot, preferred_element_type=jnp.float32,
                      precision=lax.Precision.HIGHEST)
    Sgt = within + row_suf
    grp = cd > 0
    A = jnp.where(grp, Sgt + Pd, 1.0)
    den = (jnp.sum(jnp.where(grp, cd * jnp.log(A), 0.0))
           / jnp.sum(grp.astype(jnp.float32)))
    nums = packed[:, 4 * NB:4 * NB + 2 * L]
    num = jnp.sum(nums[:, 0:L]) / jnp.sum(nums[:, L:2 * L])
    out_ref[...] = jnp.reshape(den - num, (1, 1))


def kernel(log_h, y):
    log_h = log_h.reshape(-1)
    packed = _sc_hist(log_h, y[:, 0], y[:, 1])
    out = pl.pallas_call(
        _tc_combine,
        out_shape=jax.ShapeDtypeStruct((1, 1), jnp.float32),
    )(packed)
    return out[0, 0]
